# Initial kernel scaffold; baseline (speedup 1.0000x reference)
import jax
import jax.numpy as jnp
from jax.experimental import pallas as pl


def kernel(x, edge_index, f, train_mask, y, W1, b1, W2, b2, W3, b3):
    raise NotImplementedError("write your pallas kernel here")


# speedup vs baseline: 1.0495x; 1.0495x over previous
import math
import functools

import jax
import jax.numpy as jnp
from jax.experimental import pallas as pl
from jax.experimental.pallas import tpu as pltpu

N, E, DIN, H, C = 10000, 320000, 128, 256, 40
K_PROP, ALPHA, TOPK = 10, 0.1, 70

ROW_BLK = 2000


def _mlp_body(x_ref, w1_ref, b1_ref, w2_ref, b2_ref, w3_ref, b3_ref, out_ref):
    h = jnp.maximum(jnp.dot(x_ref[...], w1_ref[...],
                            preferred_element_type=jnp.float32) + b1_ref[...], 0.0)
    h = jnp.maximum(jnp.dot(h, w2_ref[...],
                            preferred_element_type=jnp.float32) + b2_ref[...], 0.0)
    out_ref[...] = jnp.dot(h, w3_ref[...],
                           preferred_element_type=jnp.float32) + b3_ref[...]


def _mlp(x, W1, b1, W2, b2, W3, b3):
    grid = (N // ROW_BLK,)
    return pl.pallas_call(
        _mlp_body,
        grid=grid,
        in_specs=[
            pl.BlockSpec((ROW_BLK, DIN), lambda i: (i, 0)),
            pl.BlockSpec((DIN, H), lambda i: (0, 0)),
            pl.BlockSpec((H,), lambda i: (0,)),
            pl.BlockSpec((H, H), lambda i: (0, 0)),
            pl.BlockSpec((H,), lambda i: (0,)),
            pl.BlockSpec((H, C), lambda i: (0, 0)),
            pl.BlockSpec((C,), lambda i: (0,)),
        ],
        out_specs=pl.BlockSpec((ROW_BLK, C), lambda i: (i, 0)),
        out_shape=jax.ShapeDtypeStruct((N, C), jnp.float32),
    )(x, W1, b1, W2, b2, W3, b3)


def kernel(x, edge_index, f, train_mask, y, W1, b1, W2, b2, W3, b3):
    out = _mlp(x, W1, b1, W2, b2, W3, b3)

    label = f
    num_class = label.shape[1]
    total_weight = jnp.where(train_mask, 1.0, 0.0).astype(jnp.float32)
    ent_w = 1.0 - jnp.sum(-label * jnp.log(jnp.clip(label, 1e-8, None)),
                          axis=1) / math.log(num_class)
    idx = jnp.argmax(label, axis=1)
    for i in range(num_class):
        w = jnp.where(idx == i, ent_w, 0.0)
        w = jnp.where(train_mask, 0.0, w)
        vals, inds = jax.lax.top_k(w, TOPK)
        total_weight = total_weight.at[inds].set(vals)
    sm = jax.nn.softmax(out, axis=-1)
    diff = f - sm
    loss1 = jnp.sum(total_weight * jnp.sum(diff * diff, axis=1))

    src, dst = edge_index[0], edge_index[1]
    deg = jnp.zeros((N,), dtype=jnp.float32).at[dst].add(1.0)
    dinv = 1.0 / jnp.sqrt(jnp.clip(deg, 1.0, None))
    norm = dinv[src] * dinv[dst]
    z = out
    for _ in range(K_PROP):
        msg = z[src] * norm[:, None]
        agg = jax.ops.segment_sum(msg, dst, num_segments=N)
        z = (1.0 - ALPHA) * agg + ALPHA * out

    return (jax.nn.log_softmax(z, axis=1), loss1)


# speedup vs baseline: 30.3886x; 28.9563x over previous
import math
import functools

import jax
import jax.numpy as jnp
from jax import lax
from jax.experimental import pallas as pl
from jax.experimental.pallas import tpu as pltpu
from jax.experimental.pallas import tpu_sc as plsc

N, E, DIN, H, C = 10000, 320000, 128, 256, 40
K_PROP, ALPHA, TOPK = 10, 0.1, 70

NP_ = 10240
RPT = 640
DH = 32
EPT = E // 32
ECH = 2000
NCH = EPT // ECH

ROW_BLK = 2000

_mesh = plsc.VectorSubcoreMesh(core_axis_name="c", subcore_axis_name="s")


def _mlp_body(x_ref, w1_ref, b1_ref, w2_ref, b2_ref, w3_ref, b3_ref, out_ref):
    h = jnp.maximum(jnp.dot(x_ref[...], w1_ref[...],
                            preferred_element_type=jnp.float32) + b1_ref[...], 0.0)
    h = jnp.maximum(jnp.dot(h, w2_ref[...],
                            preferred_element_type=jnp.float32) + b2_ref[...], 0.0)
    out_ref[...] = jnp.dot(h, w3_ref[...],
                           preferred_element_type=jnp.float32) + b3_ref[...]


def _mlp(x, W1, b1, W2, b2, W3, b3):
    return pl.pallas_call(
        _mlp_body,
        grid=(N // ROW_BLK,),
        in_specs=[
            pl.BlockSpec((ROW_BLK, DIN), lambda i: (i, 0)),
            pl.BlockSpec((DIN, H), lambda i: (0, 0)),
            pl.BlockSpec((H,), lambda i: (0,)),
            pl.BlockSpec((H, H), lambda i: (0, 0)),
            pl.BlockSpec((H,), lambda i: (0,)),
            pl.BlockSpec((H, C), lambda i: (0, 0)),
            pl.BlockSpec((C,), lambda i: (0,)),
        ],
        out_specs=pl.BlockSpec((ROW_BLK, C), lambda i: (i, 0)),
        out_shape=jax.ShapeDtypeStruct((N, C), jnp.float32),
    )(x, W1, b1, W2, b2, W3, b3)


@functools.partial(
    pl.kernel, mesh=_mesh,
    compiler_params=pltpu.CompilerParams(use_tc_tiling_on_sc=False),
    out_type=jax.ShapeDtypeStruct((2 * NP_,), jnp.float32),
    scratch_types=[
        pltpu.VMEM((ECH,), jnp.int32),
        pltpu.VMEM((ECH,), jnp.float32),
        pltpu.VMEM((RPT,), jnp.float32),
        pltpu.VMEM_SHARED((NP_,), jnp.float32),
        pltpu.SemaphoreType.DMA,
    ],
)
def _deg_kernel(dst_hbm, deg_out, ibuf, onebuf, zbuf, acc, sem):
    cc = lax.axis_index("c")
    ss = lax.axis_index("s")
    wid = cc * 16 + ss

    def _fill(i, _):
        onebuf[pl.ds(i * 16, 16)] = jnp.full((16,), 1.0, jnp.float32)
        return 0
    lax.fori_loop(0, ECH // 16, _fill, 0)

    def _fillz(i, _):
        zbuf[pl.ds(i * 16, 16)] = jnp.zeros((16,), jnp.float32)
        return 0
    lax.fori_loop(0, RPT // 16, _fillz, 0)

    pltpu.sync_copy(zbuf, acc.at[pl.ds(ss * RPT, RPT)])
    plsc.subcore_barrier()

    def _chunk(k, _):
        base = wid * (E // 32) + k * ECH
        pltpu.sync_copy(dst_hbm.at[pl.ds(base, ECH)], ibuf)
        pltpu.sync_copy(onebuf, acc.at[ibuf], add=True)
        return 0
    lax.fori_loop(0, (E // 32) // ECH, _chunk, 0)
    plsc.subcore_barrier()

    pltpu.sync_copy(acc.at[pl.ds(ss * RPT, RPT)],
                    deg_out.at[pl.ds(cc * NP_ + ss * RPT, RPT)])


@functools.partial(
    pl.kernel, mesh=_mesh,
    compiler_params=pltpu.CompilerParams(use_tc_tiling_on_sc=False),
    out_type=(jax.ShapeDtypeStruct((2 * NP_, DH), jnp.float32),
              jax.ShapeDtypeStruct((2 * NP_, DH), jnp.float32)),
    scratch_types=[
        pltpu.VMEM((ECH,), jnp.int32),
        pltpu.VMEM((ECH,), jnp.int32),
        pltpu.VMEM((ECH, DH), jnp.float32),
        pltpu.VMEM((RPT, DH), jnp.float32),
        pltpu.VMEM((RPT, DH), jnp.float32),
        pltpu.VMEM_SHARED((NP_, DH), jnp.float32),
        pltpu.SemaphoreType.DMA,
    ],
)
def _prop_kernel(src_hbm, dst_hbm, uinit_hbm, a_hbm, c0_hbm,
                 u_hbm, s_out,
                 sbuf, dbuf, buf, abuf, cbuf, S, sem):
    cc = lax.axis_index("c")
    ss = lax.axis_index("s")
    row0 = cc * NP_ + ss * RPT
    srow = ss * RPT

    pltpu.sync_copy(a_hbm.at[pl.ds(row0, RPT)], abuf)
    pltpu.sync_copy(c0_hbm.at[pl.ds(row0, RPT)], cbuf)

    def _zero_buf(i, _):
        buf[i, pl.ds(0, 16)] = jnp.zeros((16,), jnp.float32)
        buf[i, pl.ds(16, 16)] = jnp.zeros((16,), jnp.float32)
        return 0

    pltpu.sync_copy(uinit_hbm.at[pl.ds(row0, RPT)], buf.at[pl.ds(0, RPT)])
    pltpu.sync_copy(buf.at[pl.ds(0, RPT)], u_hbm.at[pl.ds(row0, RPT)])
    lax.fori_loop(0, RPT, _zero_buf, 0)
    pltpu.sync_copy(buf.at[pl.ds(0, RPT)], S.at[pl.ds(srow, RPT)])
    plsc.subcore_barrier()

    ubase = cc * NP_

    def _scatter_phase():
        def _chunk(k, _):
            ebase = (ss * EPT) + k * ECH
            pltpu.sync_copy(src_hbm.at[pl.ds(ebase, ECH)], sbuf)
            pltpu.sync_copy(dst_hbm.at[pl.ds(ebase, ECH)], dbuf)

            def _off(i, _):
                sl = pl.ds(i * 16, 16)
                sbuf[sl] = sbuf[sl] + ubase
                return 0
            lax.fori_loop(0, ECH // 16, _off, 0)

            pltpu.async_copy(u_hbm.at[sbuf], buf, sem).wait()
            pltpu.sync_copy(buf, S.at[dbuf], add=True)
            return 0
        lax.fori_loop(0, NCH, _chunk, 0)
        plsc.subcore_barrier()

    def _iter(_k, carry):
        _scatter_phase()
        pltpu.sync_copy(S.at[pl.ds(srow, RPT)], buf.at[pl.ds(0, RPT)])

        def _upd(i, _):
            for h in (0, 16):
                sl = pl.ds(h, 16)
                buf[i, sl] = abuf[i, sl] * buf[i, sl] + cbuf[i, sl]
            return 0
        lax.fori_loop(0, RPT, _upd, 0)

        pltpu.sync_copy(buf.at[pl.ds(0, RPT)], u_hbm.at[pl.ds(row0, RPT)])
        lax.fori_loop(0, RPT, _zero_buf, 0)
        pltpu.sync_copy(buf.at[pl.ds(0, RPT)], S.at[pl.ds(srow, RPT)])
        plsc.subcore_barrier()
        return carry

    lax.fori_loop(0, K_PROP - 1, _iter, 0)
    _scatter_phase()
    pltpu.sync_copy(S.at[pl.ds(srow, RPT)], s_out.at[pl.ds(row0, RPT)])


def _prep_body(deg_ref, out_ref, dinv_ref, a_ref, c0_ref, uinit_ref):
    deg = deg_ref[0, :] + deg_ref[1, :]
    dinv = lax.rsqrt(jnp.clip(deg, 1.0, None))
    dinv_ref[...] = dinv[:, None]
    a2 = (1.0 - ALPHA) * dinv * dinv
    a_ref[0, :, :] = a2[:, None] * jnp.ones((1, DH), jnp.float32)
    a_ref[1, :, :] = a2[:, None] * jnp.ones((1, DH), jnp.float32)
    c0 = ALPHA * dinv[:, None] * out_ref[...]
    c0_ref[0, :, :] = c0[:, :DH]
    c0_ref[1, :, :] = c0[:, DH:]
    uinit_ref[0, :, :] = (1.0 / ALPHA) * c0[:, :DH]
    uinit_ref[1, :, :] = (1.0 / ALPHA) * c0[:, DH:]


def _prep(deg2, out_pad):
    return pl.pallas_call(
        _prep_body,
        out_shape=(
            jax.ShapeDtypeStruct((NP_, 1), jnp.float32),
            jax.ShapeDtypeStruct((2, NP_, DH), jnp.float32),
            jax.ShapeDtypeStruct((2, NP_, DH), jnp.float32),
            jax.ShapeDtypeStruct((2, NP_, DH), jnp.float32),
        ),
    )(deg2, out_pad)


def _fin_body(s40_ref, out_ref, dinv_ref, res_ref):
    z = (1.0 - ALPHA) * dinv_ref[...] * s40_ref[...] + ALPHA * out_ref[...]
    m = jnp.max(z, axis=1, keepdims=True)
    zs = z - m
    res_ref[...] = zs - jnp.log(jnp.sum(jnp.exp(zs), axis=1, keepdims=True))


def _finish(s40, out, dinv):
    return pl.pallas_call(
        _fin_body,
        grid=(N // ROW_BLK,),
        in_specs=[
            pl.BlockSpec((ROW_BLK, C), lambda i: (i, 0)),
            pl.BlockSpec((ROW_BLK, C), lambda i: (i, 0)),
            pl.BlockSpec((ROW_BLK, 1), lambda i: (i, 0)),
        ],
        out_specs=pl.BlockSpec((ROW_BLK, C), lambda i: (i, 0)),
        out_shape=jax.ShapeDtypeStruct((N, C), jnp.float32),
    )(s40, out, dinv)


def kernel(x, edge_index, f, train_mask, y, W1, b1, W2, b2, W3, b3):
    out = _mlp(x, W1, b1, W2, b2, W3, b3)

    src = edge_index[0]
    dst = edge_index[1]

    deg2 = _deg_kernel(dst).reshape(2, NP_)
    out_pad = jnp.pad(out, ((0, NP_ - N), (0, 2 * DH - C)))
    dinv, a_arr, c0_arr, uinit = _prep(deg2, out_pad)

    _uw, s_out = _prop_kernel(src, dst,
                              uinit.reshape(2 * NP_, DH),
                              a_arr.reshape(2 * NP_, DH),
                              c0_arr.reshape(2 * NP_, DH))
    s_out = s_out.reshape(2, NP_, DH)
    s40 = jnp.concatenate([s_out[0, :N, :], s_out[1, :N, :C - DH]], axis=1)
    res = _finish(s40, out, dinv[:N, :])

    label = f
    num_class = label.shape[1]
    total_weight = jnp.where(train_mask, 1.0, 0.0).astype(jnp.float32)
    ent_w = 1.0 - jnp.sum(-label * jnp.log(jnp.clip(label, 1e-8, None)),
                          axis=1) / math.log(num_class)
    idx = jnp.argmax(label, axis=1)
    for i in range(num_class):
        w = jnp.where(idx == i, ent_w, 0.0)
        w = jnp.where(train_mask, 0.0, w)
        vals, inds = jax.lax.top_k(w, TOPK)
        total_weight = total_weight.at[inds].set(vals)
    sm = jax.nn.softmax(out, axis=-1)
    diff = f - sm
    loss1 = jnp.sum(total_weight * jnp.sum(diff * diff, axis=1))

    return (res, loss1)
